# Initial kernel scaffold; baseline (speedup 1.0000x reference)
import jax
import jax.numpy as jnp
from jax.experimental import pallas as pl


def kernel(vert, vol):
    raise NotImplementedError("write your pallas kernel here")


# speedup vs baseline: 3.1380x; 3.1380x over previous
import functools

import jax
import jax.numpy as jnp
from jax import lax
from jax.experimental import pallas as pl
from jax.experimental.pallas import tpu as pltpu
from jax.experimental.pallas import tpu_sc as plsc

X, Y, Z = 112, 224, 160
C = 8
ZQ = Z // 4
NROWS = X * Y * ZQ
V = 100000
NW = 32
GP16 = 16
GPW = 196
VPW = GPW * GP16
VPAD = NW * VPW

_F32 = jnp.float32
_I32 = jnp.int32


def _cr_weights(u):
    u2 = u * u
    u3 = u2 * u
    w0 = 0.5 * (-u3 + 2.0 * u2 - u)
    w1 = 0.5 * (3.0 * u3 - 5.0 * u2 + 2.0)
    w2 = 0.5 * (-3.0 * u3 + 4.0 * u2 + u)
    w3 = 0.5 * (u3 - u2)
    return (w0, w1, w2, w3)


def _interp_body(table, vertf, out_hbm, vert_v, idx_v0, idx_v1, dat_v0,
                 dat_v1, out_v, sem0, sem1):
    wid = lax.axis_index("s") * 2 + lax.axis_index("c")
    lanes = lax.iota(_I32, 16)
    lanes3 = lanes * 3
    lanes8 = lanes * 8

    pltpu.sync_copy(vertf.at[pl.ds(wid * (VPW * 3), VPW * 3)], vert_v)

    minb = 1.0 + 1e-5

    def load_coords(g):
        b = g * (GP16 * 3)
        vx = plsc.load_gather(vert_v, [lanes3 + b])
        vy = plsc.load_gather(vert_v, [lanes3 + (b + 1)])
        vz = plsc.load_gather(vert_v, [lanes3 + (b + 2)])
        vx = jnp.clip(vx, minb, X - 2.0 - 1e-5)
        vy = jnp.clip(vy, minb, Y - 2.0 - 1e-5)
        vz = jnp.clip(vz, minb, Z - 2.0 - 1e-5)
        xi = vx.astype(_I32)
        yi = vy.astype(_I32)
        zi = vz.astype(_I32)
        ux = vx - xi.astype(_F32)
        uy = vy - yi.astype(_F32)
        uz = vz - zi.astype(_F32)
        return xi - 1, yi - 1, zi - 1, ux, uy, uz

    def build_and_fire(g, idx_v, dat_v, sem):
        x0, y0, z0, _, _, _ = load_coords(g)
        q0 = lax.shift_right_arithmetic(z0, 2)
        q1 = jnp.minimum(q0 + 1, ZQ - 1)
        a = x0 * (Y * ZQ) + y0 * ZQ
        aq0 = a + q0
        aq1 = a + q1
        for i in range(4):
            for j in range(4):
                off = i * (Y * ZQ) + j * ZQ
                s = (i * 4 + j) * 2
                idx_v[s >> 3, pl.ds((s & 7) * 16, 16)] = aq0 + off
                s += 1
                idx_v[s >> 3, pl.ds((s & 7) * 16, 16)] = aq1 + off
        for q in range(4):
            pltpu.async_copy(table.at[idx_v.at[q]], dat_v.at[q], sem)

    def wait_and_compute(g, idx_v, dat_v, sem):
        for q in range(4):
            pltpu.make_async_copy(table.at[idx_v.at[q]], dat_v.at[q],
                                  sem).wait()
        _, _, z0, ux, uy, uz = load_coords(g)
        wx = _cr_weights(ux)
        wy = _cr_weights(uy)
        wz = _cr_weights(uz)
        q0 = lax.shift_right_arithmetic(z0, 2)
        rows_k = []
        elem_k = []
        for k in range(4):
            zk = z0 + k
            qk = lax.shift_right_arithmetic(zk, 2)
            rows_k.append((qk - q0) * 16 + lanes)
            elem_k.append((zk & 3) * 8)
        acc = [jnp.zeros((16,), _F32) for _ in range(C)]
        for i in range(4):
            for j in range(4):
                wij = wx[i] * wy[j]
                fr_base = (i * 4 + j) * 32
                for k in range(4):
                    w = wij * wz[k]
                    flat = rows_k[k] + fr_base
                    qrow = lax.shift_right_arithmetic(flat, 7)
                    col = flat & 127
                    for c in range(C):
                        val = plsc.load_gather(dat_v,
                                               [qrow, col, elem_k[k] + c])
                        acc[c] = acc[c] + w * val
        ob = g * (GP16 * C)
        for c in range(C):
            plsc.store_scatter(out_v, [lanes8 + (ob + c)], acc[c])

    build_and_fire(0, idx_v0, dat_v0, sem0)

    def outer(it, carry):
        g = it * 2
        build_and_fire(g + 1, idx_v1, dat_v1, sem1)
        wait_and_compute(g, idx_v0, dat_v0, sem0)

        @pl.when(g + 2 < GPW)
        def _():
            build_and_fire(g + 2, idx_v0, dat_v0, sem0)

        wait_and_compute(g + 1, idx_v1, dat_v1, sem1)
        return carry

    lax.fori_loop(0, GPW // 2, outer, 0)
    pltpu.sync_copy(out_v, out_hbm.at[pl.ds(wid * (VPW * C), VPW * C)])


@jax.jit
def kernel(vert, vol):
    table = jnp.transpose(vol[0], (1, 2, 3, 0)).reshape(NROWS, 32)
    vp = jnp.pad(vert[0], ((0, VPAD - V), (0, 0)), constant_values=2.0)
    vertf = vp.reshape(VPAD * 3)

    mesh = plsc.VectorSubcoreMesh(core_axis_name="c", subcore_axis_name="s")
    run = pl.kernel(
        _interp_body,
        mesh=mesh,
        compiler_params=pltpu.CompilerParams(needs_layout_passes=False,
                                              use_tc_tiling_on_sc=False),
        out_type=jax.ShapeDtypeStruct((VPAD * C,), _F32),
        scratch_types=[
            pltpu.VMEM((VPW * 3,), _F32),
            pltpu.VMEM((4, 128), _I32),
            pltpu.VMEM((4, 128), _I32),
            pltpu.VMEM((4, 128, 32), _F32),
            pltpu.VMEM((4, 128, 32), _F32),
            pltpu.VMEM((VPW * C,), _F32),
            pltpu.SemaphoreType.DMA,
            pltpu.SemaphoreType.DMA,
        ],
    )
    out = run(table, vertf)
    return out.reshape(VPAD, C)[:V][None]


# speedup vs baseline: 4.0590x; 1.2935x over previous
import functools

import jax
import jax.numpy as jnp
from jax import lax
from jax.experimental import pallas as pl
from jax.experimental.pallas import tpu as pltpu
from jax.experimental.pallas import tpu_sc as plsc

X, Y, Z = 112, 224, 160
C = 8
ZQ = Z // 4
NROWS = X * Y * ZQ
V = 100000
NW = 32
GP16 = 16
GPW = 196
VPW = GPW * GP16
VPAD = NW * VPW

_F32 = jnp.float32
_I32 = jnp.int32


def _cr_weights(u):
    u2 = u * u
    u3 = u2 * u
    w0 = 0.5 * (-u3 + 2.0 * u2 - u)
    w1 = 0.5 * (3.0 * u3 - 5.0 * u2 + 2.0)
    w2 = 0.5 * (-3.0 * u3 + 4.0 * u2 + u)
    w3 = 0.5 * (u3 - u2)
    return (w0, w1, w2, w3)


def _interp_body(table, vertf, out_hbm, vert_v, idx_v0, idx_v1, dat_v0,
                 dat_v1, out_v, sem0, sem1):
    wid = lax.axis_index("s") * 2 + lax.axis_index("c")
    lanes = lax.iota(_I32, 16)
    lanes3 = lanes * 3
    lanes8 = lanes * 8

    pltpu.sync_copy(vertf.at[pl.ds(wid * (VPW * 3), VPW * 3)], vert_v)

    minb = 1.0 + 1e-5

    def load_coords(g):
        b = g * (GP16 * 3)
        vx = plsc.load_gather(vert_v, [lanes3 + b])
        vy = plsc.load_gather(vert_v, [lanes3 + (b + 1)])
        vz = plsc.load_gather(vert_v, [lanes3 + (b + 2)])
        vx = jnp.clip(vx, minb, X - 2.0 - 1e-5)
        vy = jnp.clip(vy, minb, Y - 2.0 - 1e-5)
        vz = jnp.clip(vz, minb, Z - 2.0 - 1e-5)
        xi = vx.astype(_I32)
        yi = vy.astype(_I32)
        zi = vz.astype(_I32)
        ux = vx - xi.astype(_F32)
        uy = vy - yi.astype(_F32)
        uz = vz - zi.astype(_F32)
        return xi - 1, yi - 1, zi - 1, ux, uy, uz

    def build_and_fire(g, idx_v, dat_v, sem):
        x0, y0, z0, _, _, _ = load_coords(g)
        q0 = lax.shift_right_arithmetic(z0, 2)
        q1 = jnp.minimum(q0 + 1, ZQ - 1)
        a = x0 * (Y * ZQ) + y0 * ZQ
        aq0 = a + q0
        aq1 = a + q1
        for i in range(4):
            for j in range(4):
                off = i * (Y * ZQ) + j * ZQ
                s = (i * 4 + j) * 2
                idx_v[s >> 3, pl.ds((s & 7) * 16, 16)] = aq0 + off
                s += 1
                idx_v[s >> 3, pl.ds((s & 7) * 16, 16)] = aq1 + off
        for q in range(4):
            pltpu.async_copy(table.at[idx_v.at[q]], dat_v.at[q], sem)

    def wait_and_compute(g, idx_v, dat_v, sem):
        for q in range(4):
            pltpu.make_async_copy(table.at[idx_v.at[q]], dat_v.at[q],
                                  sem).wait()
        _, _, z0, ux, uy, uz = load_coords(g)
        wx = _cr_weights(ux)
        wy = _cr_weights(uy)
        wz = _cr_weights(uz)
        q0 = lax.shift_right_arithmetic(z0, 2)
        rows_k = []
        elem_k = []
        for k in range(4):
            zk = z0 + k
            qk = lax.shift_right_arithmetic(zk, 2)
            rows_k.append((qk - q0) * 16 + lanes)
            elem_k.append(zk & 3)
        acc = [jnp.zeros((16,), _F32) for _ in range(C)]
        for i in range(4):
            for j in range(4):
                wij = wx[i] * wy[j]
                fr_base = (i * 4 + j) * 32
                for k in range(4):
                    w = wij * wz[k]
                    flat = rows_k[k] + fr_base
                    qrow = lax.shift_right_arithmetic(flat, 7)
                    col = flat & 127
                    for c in range(C):
                        val = plsc.load_gather(dat_v,
                                               [qrow, col, elem_k[k] + c * 4])
                        acc[c] = acc[c] + w * val
        ob = g * (GP16 * C)
        for c in range(C):
            plsc.store_scatter(out_v, [lanes8 + (ob + c)], acc[c])

    build_and_fire(0, idx_v0, dat_v0, sem0)

    def outer(it, carry):
        g = it * 2
        build_and_fire(g + 1, idx_v1, dat_v1, sem1)
        wait_and_compute(g, idx_v0, dat_v0, sem0)

        @pl.when(g + 2 < GPW)
        def _():
            build_and_fire(g + 2, idx_v0, dat_v0, sem0)

        wait_and_compute(g + 1, idx_v1, dat_v1, sem1)
        return carry

    lax.fori_loop(0, GPW // 2, outer, 0)
    pltpu.sync_copy(out_v, out_hbm.at[pl.ds(wid * (VPW * C), VPW * C)])


BAND = 14
NXC = Y // BAND
TASKS = X * NXC
TPW = TASKS // NW
CROWS = BAND * ZQ


def _relayout_sc_body(vol_ref, tab_ref, in_v0, in_v1, out_v0, out_v1,
                      sem_i0, sem_i1, sem_o0, sem_o1):
    wid = lax.axis_index("s") * 2 + lax.axis_index("c")
    lanes = lax.iota(_I32, 16)
    pat = lax.shift_right_arithmetic(lanes, 2) * 32 + (lanes & 3)
    t0 = wid * TPW

    def task_xy(t):
        return lax.shift_right_arithmetic(t, 4), (t & (NXC - 1)) * BAND

    def fire_in(t, in_v, sem):
        x, y0 = task_xy(t)
        for c in range(C):
            pltpu.async_copy(vol_ref.at[0, c, x, pl.ds(y0, BAND)],
                             in_v.at[c], sem)

    def wait_in(t, in_v, sem):
        x, y0 = task_xy(t)
        for c in range(C):
            pltpu.make_async_copy(vol_ref.at[0, c, x, pl.ds(y0, BAND)],
                                  in_v.at[c], sem).wait()

    def out_slice(t):
        x, y0 = task_xy(t)
        return tab_ref.at[pl.ds((x * Y + y0) * ZQ, CROWS)]

    def compute(in_v, out_v):
        def yy_body(yy, _):
            def gz_body(gz, _):
                rowc = yy * 40 + gz * 4
                for c in range(C):
                    v = in_v[c, yy, pl.ds(gz * 16, 16)]
                    pos = pat + (rowc * 32 + c * 4)
                    plsc.store_scatter(
                        out_v, [lax.shift_right_arithmetic(pos, 5),
                                pos & 31], v)
                return 0
            return lax.fori_loop(0, ZQ // 4, gz_body, 0)
        lax.fori_loop(0, BAND, yy_body, 0)

    bufs = ((in_v0, out_v0, sem_i0, sem_o0), (in_v1, out_v1, sem_i1, sem_o1))
    fire_in(t0, in_v0, sem_i0)
    fire_in(t0 + 1, in_v1, sem_i1)

    def step(i, carry):
        for p in range(2):
            t = t0 + i * 2 + p
            in_v, out_v, sem_i, sem_o = bufs[p]
            wait_in(t, in_v, sem_i)

            @pl.when(i > 0)
            def _():
                pltpu.make_async_copy(out_v, out_slice(t - 2), sem_o).wait()

            compute(in_v, out_v)
            pltpu.async_copy(out_v, out_slice(t), sem_o)

            @pl.when(i * 2 + p + 2 < TPW)
            def _():
                fire_in(t + 2, in_v, sem_i)
        return carry

    lax.fori_loop(0, TPW // 2, step, 0)
    for p in range(2):
        t = t0 + TPW - 2 + p
        _, out_v, _, sem_o = bufs[p]
        pltpu.make_async_copy(out_v, out_slice(t), sem_o).wait()


def _make_table(vol):
    mesh = plsc.VectorSubcoreMesh(core_axis_name="c", subcore_axis_name="s")
    run = pl.kernel(
        _relayout_sc_body,
        mesh=mesh,
        compiler_params=pltpu.CompilerParams(needs_layout_passes=False,
                                             use_tc_tiling_on_sc=False),
        out_type=jax.ShapeDtypeStruct((NROWS, 32), _F32),
        scratch_types=[
            pltpu.VMEM((C, BAND, Z), _F32),
            pltpu.VMEM((C, BAND, Z), _F32),
            pltpu.VMEM((CROWS, 32), _F32),
            pltpu.VMEM((CROWS, 32), _F32),
            pltpu.SemaphoreType.DMA,
            pltpu.SemaphoreType.DMA,
            pltpu.SemaphoreType.DMA,
            pltpu.SemaphoreType.DMA,
        ],
    )
    return run(vol)


@jax.jit
def kernel(vert, vol):
    table = _make_table(vol)
    vp = jnp.pad(vert[0], ((0, VPAD - V), (0, 0)), constant_values=2.0)
    vertf = vp.reshape(VPAD * 3)

    mesh = plsc.VectorSubcoreMesh(core_axis_name="c", subcore_axis_name="s")
    run = pl.kernel(
        _interp_body,
        mesh=mesh,
        compiler_params=pltpu.CompilerParams(needs_layout_passes=False,
                                              use_tc_tiling_on_sc=False),
        out_type=jax.ShapeDtypeStruct((VPAD * C,), _F32),
        scratch_types=[
            pltpu.VMEM((VPW * 3,), _F32),
            pltpu.VMEM((4, 128), _I32),
            pltpu.VMEM((4, 128), _I32),
            pltpu.VMEM((4, 128, 32), _F32),
            pltpu.VMEM((4, 128, 32), _F32),
            pltpu.VMEM((VPW * C,), _F32),
            pltpu.SemaphoreType.DMA,
            pltpu.SemaphoreType.DMA,
        ],
    )
    out = run(table, vertf)
    return out.reshape(VPAD, C)[:V][None]


# speedup vs baseline: 4.0599x; 1.0002x over previous
import functools

import jax
import jax.numpy as jnp
from jax import lax
from jax.experimental import pallas as pl
from jax.experimental.pallas import tpu as pltpu
from jax.experimental.pallas import tpu_sc as plsc

X, Y, Z = 112, 224, 160
C = 8
ZQ = Z // 4
NROWS = X * Y * ZQ
V = 100000
NW = 32
GP16 = 16
GPW = 196
VPW = GPW * GP16
VPAD = NW * VPW

_F32 = jnp.float32
_I32 = jnp.int32


def _cr_weights(u):
    u2 = u * u
    u3 = u2 * u
    w0 = 0.5 * (-u3 + 2.0 * u2 - u)
    w1 = 0.5 * (3.0 * u3 - 5.0 * u2 + 2.0)
    w2 = 0.5 * (-3.0 * u3 + 4.0 * u2 + u)
    w3 = 0.5 * (u3 - u2)
    return (w0, w1, w2, w3)


def _interp_body(table, vertf, out_hbm, vert_v, idx_v0, idx_v1, dat_v0,
                 dat_v1, out_v, sem0, sem1):
    wid = lax.axis_index("s") * 2 + lax.axis_index("c")
    lanes = lax.iota(_I32, 16)
    lanes3 = lanes * 3
    lanes8 = lanes * 8

    pltpu.sync_copy(vertf.at[pl.ds(wid * (VPW * 3), VPW * 3)], vert_v)

    minb = 1.0 + 1e-5

    def load_coords(g):
        b = g * (GP16 * 3)
        vx = plsc.load_gather(vert_v, [lanes3 + b])
        vy = plsc.load_gather(vert_v, [lanes3 + (b + 1)])
        vz = plsc.load_gather(vert_v, [lanes3 + (b + 2)])
        vx = jnp.clip(vx, minb, X - 2.0 - 1e-5)
        vy = jnp.clip(vy, minb, Y - 2.0 - 1e-5)
        vz = jnp.clip(vz, minb, Z - 2.0 - 1e-5)
        xi = vx.astype(_I32)
        yi = vy.astype(_I32)
        zi = vz.astype(_I32)
        ux = vx - xi.astype(_F32)
        uy = vy - yi.astype(_F32)
        uz = vz - zi.astype(_F32)
        return xi - 1, yi - 1, zi - 1, ux, uy, uz

    def build_and_fire(g, idx_v, dat_v, sem):
        x0, y0, z0, _, _, _ = load_coords(g)
        q0 = lax.shift_right_arithmetic(z0, 2)
        q1 = jnp.minimum(q0 + 1, ZQ - 1)
        a = x0 * (Y * ZQ) + y0 * ZQ
        aq0 = a + q0
        aq1 = a + q1
        for i in range(4):
            for j in range(4):
                off = i * (Y * ZQ) + j * ZQ
                s = (i * 4 + j) * 2
                idx_v[s >> 3, pl.ds((s & 7) * 16, 16)] = aq0 + off
                s += 1
                idx_v[s >> 3, pl.ds((s & 7) * 16, 16)] = aq1 + off
        for q in range(4):
            pltpu.async_copy(table.at[idx_v.at[q]], dat_v.at[q], sem)

    def wait_and_compute(g, idx_v, dat_v, sem):
        for q in range(4):
            pltpu.make_async_copy(table.at[idx_v.at[q]], dat_v.at[q],
                                  sem).wait()
        _, _, z0, ux, uy, uz = load_coords(g)
        wx = _cr_weights(ux)
        wy = _cr_weights(uy)
        wz = _cr_weights(uz)
        q0 = lax.shift_right_arithmetic(z0, 2)
        rows_k = []
        elem_k = []
        for k in range(4):
            zk = z0 + k
            qk = lax.shift_right_arithmetic(zk, 2)
            rows_k.append((qk - q0) * 16 + lanes)
            elem_k.append(zk & 3)
        acc = [jnp.zeros((16,), _F32) for _ in range(C)]
        for i in range(4):
            for j in range(4):
                wij = wx[i] * wy[j]
                fr_base = (i * 4 + j) * 32
                for k in range(4):
                    w = wij * wz[k]
                    flat = rows_k[k] + fr_base
                    qrow = lax.shift_right_arithmetic(flat, 7)
                    col = flat & 127
                    for c in range(C):
                        val = plsc.load_gather(dat_v,
                                               [qrow, col, elem_k[k] + c * 4])
                        acc[c] = acc[c] + w * val
        ob = g * (GP16 * C)
        for c in range(C):
            plsc.store_scatter(out_v, [lanes8 + (ob + c)], acc[c])

    build_and_fire(0, idx_v0, dat_v0, sem0)

    def outer(it, carry):
        g = it * 2
        build_and_fire(g + 1, idx_v1, dat_v1, sem1)
        wait_and_compute(g, idx_v0, dat_v0, sem0)

        @pl.when(g + 2 < GPW)
        def _():
            build_and_fire(g + 2, idx_v0, dat_v0, sem0)

        wait_and_compute(g + 1, idx_v1, dat_v1, sem1)
        return carry

    lax.fori_loop(0, GPW // 2, outer, 0)
    pltpu.sync_copy(out_v, out_hbm.at[pl.ds(wid * (VPW * C), VPW * C)])


BAND = 14
NXC = Y // BAND
TASKS = X * NXC
TPW = TASKS // NW
CROWS = BAND * ZQ


def _relayout_sc_body(vol_ref, tab_ref, in_v0, in_v1, out_v0, out_v1,
                      sem_i0, sem_i1, sem_o0, sem_o1):
    wid = lax.axis_index("s") * 2 + lax.axis_index("c")
    lanes = lax.iota(_I32, 16)
    t0 = wid * TPW

    def task_xy(t):
        return lax.shift_right_arithmetic(t, 4), (t & (NXC - 1)) * BAND

    def fire_in(t, in_v, sem):
        x, y0 = task_xy(t)
        for c in range(C):
            src = vol_ref.at[pl.ds(((c * X + x) * Y + y0) * Z, BAND * Z)]
            pltpu.async_copy(src, in_v.at[c], sem)

    def wait_in(t, in_v, sem):
        x, y0 = task_xy(t)
        for c in range(C):
            src = vol_ref.at[pl.ds(((c * X + x) * Y + y0) * Z, BAND * Z)]
            pltpu.make_async_copy(src, in_v.at[c], sem).wait()

    def out_slice(t):
        x, y0 = task_xy(t)
        return tab_ref.at[pl.ds((x * Y + y0) * ZQ, CROWS)]

    rowp = lax.shift_right_arithmetic(lanes, 2)
    colp = lanes & 3

    def compute(in_v, out_v):
        def yy_body(yy, _):
            def gz_body(gz, _):
                g = yy * (ZQ // 4) + gz
                rowc = yy * ZQ + gz * 4
                for c in range(C):
                    v = in_v[c, pl.ds(g * 16, 16)]
                    plsc.store_scatter(out_v, [rowp + rowc, colp + c * 4], v)
                return 0
            return lax.fori_loop(0, ZQ // 4, gz_body, 0)
        lax.fori_loop(0, BAND, yy_body, 0)

    bufs = ((in_v0, out_v0, sem_i0, sem_o0), (in_v1, out_v1, sem_i1, sem_o1))
    fire_in(t0, in_v0, sem_i0)
    fire_in(t0 + 1, in_v1, sem_i1)

    def step(i, carry):
        for p in range(2):
            t = t0 + i * 2 + p
            in_v, out_v, sem_i, sem_o = bufs[p]
            wait_in(t, in_v, sem_i)

            @pl.when(i > 0)
            def _():
                pltpu.make_async_copy(out_v, out_slice(t - 2), sem_o).wait()

            compute(in_v, out_v)
            pltpu.async_copy(out_v, out_slice(t), sem_o)

            @pl.when(i * 2 + p + 2 < TPW)
            def _():
                fire_in(t + 2, in_v, sem_i)
        return carry

    lax.fori_loop(0, TPW // 2, step, 0)
    for p in range(2):
        t = t0 + TPW - 2 + p
        _, out_v, _, sem_o = bufs[p]
        pltpu.make_async_copy(out_v, out_slice(t), sem_o).wait()


def _make_table(vol):
    mesh = plsc.VectorSubcoreMesh(core_axis_name="c", subcore_axis_name="s")
    run = pl.kernel(
        _relayout_sc_body,
        mesh=mesh,
        compiler_params=pltpu.CompilerParams(needs_layout_passes=False,
                                             use_tc_tiling_on_sc=False),
        out_type=jax.ShapeDtypeStruct((NROWS, 32), _F32),
        scratch_types=[
            pltpu.VMEM((C, BAND * Z), _F32),
            pltpu.VMEM((C, BAND * Z), _F32),
            pltpu.VMEM((CROWS, 32), _F32),
            pltpu.VMEM((CROWS, 32), _F32),
            pltpu.SemaphoreType.DMA,
            pltpu.SemaphoreType.DMA,
            pltpu.SemaphoreType.DMA,
            pltpu.SemaphoreType.DMA,
        ],
    )
    return run(vol.reshape(C * X * Y * Z))


@jax.jit
def kernel(vert, vol):
    table = _make_table(vol)
    vp = jnp.pad(vert[0], ((0, VPAD - V), (0, 0)), constant_values=2.0)
    vertf = vp.reshape(VPAD * 3)

    mesh = plsc.VectorSubcoreMesh(core_axis_name="c", subcore_axis_name="s")
    run = pl.kernel(
        _interp_body,
        mesh=mesh,
        compiler_params=pltpu.CompilerParams(needs_layout_passes=False,
                                              use_tc_tiling_on_sc=False),
        out_type=jax.ShapeDtypeStruct((VPAD * C,), _F32),
        scratch_types=[
            pltpu.VMEM((VPW * 3,), _F32),
            pltpu.VMEM((4, 128), _I32),
            pltpu.VMEM((4, 128), _I32),
            pltpu.VMEM((4, 128, 32), _F32),
            pltpu.VMEM((4, 128, 32), _F32),
            pltpu.VMEM((VPW * C,), _F32),
            pltpu.SemaphoreType.DMA,
            pltpu.SemaphoreType.DMA,
        ],
    )
    out = run(table, vertf)
    return out.reshape(VPAD, C)[:V][None]
